# Initial kernel scaffold; baseline (speedup 1.0000x reference)
import jax
import jax.numpy as jnp
from jax.experimental import pallas as pl


def kernel(sampled_lidar_list, raw_lidar_list):
    raise NotImplementedError("write your pallas kernel here")


# speedup vs baseline: 5.2948x; 5.2948x over previous
import functools

import jax
import jax.numpy as jnp
from jax.experimental import pallas as pl
from jax.experimental.pallas import tpu as pltpu

_B = 4
_NS = 2048
_NR = 8192
_BLK = 512
_NJ = _NR // _BLK


def _loss_kernel(s_ref, r_ref, out_ref, acc_ref):
    b = pl.program_id(0)
    j = pl.program_id(1)

    @pl.when(jnp.logical_and(b == 0, j == 0))
    def _init_out():
        out_ref[0, 0] = 0.0

    sx = s_ref[0, 0:1, :]
    sy = s_ref[0, 1:2, :]
    sz = s_ref[0, 2:3, :]
    rx = r_ref[0, :, 0:1]
    ry = r_ref[0, :, 1:2]
    rz = r_ref[0, :, 2:3]

    dx = rx - sx
    dy = ry - sy
    dz = rz - sz
    d2 = dx * dx + dy * dy + dz * dz

    raw_min = jnp.min(d2, axis=1, keepdims=True)
    raw_sum = jnp.sum(jnp.sqrt(raw_min))
    out_ref[0, 0] += raw_sum * (5.0 / (_B * _NR))

    samp_min = jnp.min(d2, axis=0, keepdims=True)

    @pl.when(j == 0)
    def _init_acc():
        acc_ref[...] = samp_min

    @pl.when(j != 0)
    def _acc():
        acc_ref[...] = jnp.minimum(acc_ref[...], samp_min)

    @pl.when(j == _NJ - 1)
    def _finish_cloud():
        sq = jnp.sqrt(acc_ref[...])
        lf = jnp.sum(sq) * (1.0 / _NS)
        lm = jnp.max(sq)
        out_ref[0, 0] += (lf + lm) * (1.0 / _B)


@functools.partial(jax.jit, static_argnames=("interpret",))
def kernel(sampled_lidar_list, raw_lidar_list, interpret=False):
    s = jnp.transpose(sampled_lidar_list[:, :, 0:3], (0, 2, 1))
    r = raw_lidar_list[:, :, 0:3]
    out = pl.pallas_call(
        _loss_kernel,
        grid=(_B, _NJ),
        in_specs=[
            pl.BlockSpec((1, 3, _NS), lambda b, j: (b, 0, 0)),
            pl.BlockSpec((1, _BLK, 3), lambda b, j: (b, j, 0)),
        ],
        out_specs=pl.BlockSpec(
            (1, 1), lambda b, j: (0, 0), memory_space=pltpu.SMEM
        ),
        out_shape=jax.ShapeDtypeStruct((1, 1), jnp.float32),
        scratch_shapes=[pltpu.VMEM((1, _NS), jnp.float32)],
        interpret=interpret,
    )(s, r)
    return out[0, 0]


# speedup vs baseline: 8.3538x; 1.5777x over previous
import functools

import jax
import jax.numpy as jnp
from jax.experimental import pallas as pl
from jax.experimental.pallas import tpu as pltpu

_B = 4
_NS = 2048
_NR = 8192
_BLK = 512
_NJ = _NR // _BLK


def _loss_kernel(s_ref, r_ref, out_ref, acc_ref):
    b = pl.program_id(0)
    j = pl.program_id(1)

    @pl.when(jnp.logical_and(b == 0, j == 0))
    def _init_out():
        out_ref[0, 0] = 0.0

    s = s_ref[0]
    rb = r_ref[0]
    g = jax.lax.dot_general(
        rb, s, (((1,), (0,)), ((), ())),
        preferred_element_type=jnp.float32,
    )
    r2 = jnp.sum(rb * rb, axis=1, keepdims=True)
    s2 = jnp.sum(s * s, axis=0, keepdims=True)
    d2 = (r2 + s2) - (g + g)

    raw_min = jnp.min(d2, axis=1, keepdims=True)
    raw_sum = jnp.sum(jnp.sqrt(jnp.maximum(raw_min, 0.0)))
    out_ref[0, 0] += raw_sum * (5.0 / (_B * _NR))

    samp_min = jnp.min(d2, axis=0, keepdims=True)

    @pl.when(j == 0)
    def _init_acc():
        acc_ref[...] = samp_min

    @pl.when(j != 0)
    def _acc():
        acc_ref[...] = jnp.minimum(acc_ref[...], samp_min)

    @pl.when(j == _NJ - 1)
    def _finish_cloud():
        sq = jnp.sqrt(jnp.maximum(acc_ref[...], 0.0))
        lf = jnp.sum(sq) * (1.0 / _NS)
        lm = jnp.max(sq)
        out_ref[0, 0] += (lf + lm) * (1.0 / _B)


@functools.partial(jax.jit, static_argnames=("interpret",))
def kernel(sampled_lidar_list, raw_lidar_list, interpret=False):
    s = jnp.transpose(sampled_lidar_list[:, :, 0:3], (0, 2, 1))
    r = raw_lidar_list[:, :, 0:3]
    out = pl.pallas_call(
        _loss_kernel,
        grid=(_B, _NJ),
        in_specs=[
            pl.BlockSpec((1, 3, _NS), lambda b, j: (b, 0, 0)),
            pl.BlockSpec((1, _BLK, 3), lambda b, j: (b, j, 0)),
        ],
        out_specs=pl.BlockSpec(
            (1, 1), lambda b, j: (0, 0), memory_space=pltpu.SMEM
        ),
        out_shape=jax.ShapeDtypeStruct((1, 1), jnp.float32),
        scratch_shapes=[pltpu.VMEM((1, _NS), jnp.float32)],
        interpret=interpret,
    )(s, r)
    return out[0, 0]
